# baseline (speedup 1.0000x reference)
import functools

import jax
import jax.numpy as jnp
from jax import lax
from jax.experimental import pallas as pl
from jax.experimental.pallas import tpu as pltpu
from jax.experimental.pallas import tpu_sc as plsc

DIM = 1024
INTER = 512
E = 8
T = 2048
BM = 128
P = 4096 + E * BM
NBLK = P // BM
NW = 32
TPW = T // NW
CT = 32
NCHK = TPW // CT
TB = 512


def _silu(g):
    return g * (1.0 / (1.0 + jnp.exp(-g)))


def _gate_body(x_ref, wg_ref, bg_ref, ws1_ref, bs1_ref, ws3_ref, bs3_ref,
               ws2_ref, bs2_ref, pos_ref, wts_ref, bmap_ref, z_ref):
    dn = (((1,), (1,)), ((), ()))
    x = x_ref[...]
    scores = lax.dot_general(x, wg_ref[...], dn,
                             preferred_element_type=jnp.float32)
    scores = scores + bg_ref[...]
    smax = jnp.max(scores, axis=1, keepdims=True)
    ex = jnp.exp(scores - smax)
    p = ex / jnp.sum(ex, axis=1, keepdims=True)
    iota8 = lax.broadcasted_iota(jnp.int32, (T, E), 1)
    m1 = jnp.max(p, axis=1, keepdims=True)
    i1 = jnp.min(jnp.where(p == m1, iota8, E), axis=1, keepdims=True)
    pm = jnp.where(iota8 == i1, -jnp.inf, p)
    m2 = jnp.max(pm, axis=1, keepdims=True)
    i2 = jnp.min(jnp.where(pm == m2, iota8, E), axis=1, keepdims=True)

    one1 = (iota8 == i1).astype(jnp.float32)
    one2 = (iota8 == i2).astype(jnp.float32)

    r_i = lax.broadcasted_iota(jnp.int32, (T, T), 0)
    c_i = lax.broadcasted_iota(jnp.int32, (T, T), 1)
    ls = (c_i < r_i).astype(jnp.float32)
    rank1 = lax.dot_general(ls, one1, (((1,), (0,)), ((), ())),
                            preferred_element_type=jnp.float32)
    cnt1 = jnp.sum(one1, axis=0, keepdims=True)
    rank2 = lax.dot_general(ls, one2, (((1,), (0,)), ((), ())),
                            preferred_element_type=jnp.float32) + cnt1
    counts = cnt1 + jnp.sum(one2, axis=0, keepdims=True)

    pc = jnp.floor((counts + (BM - 1)) * (1.0 / BM)).astype(jnp.float32) * BM
    rj = lax.broadcasted_iota(jnp.int32, (E, E), 0)
    ce = lax.broadcasted_iota(jnp.int32, (E, E), 1)
    u8 = (rj < ce).astype(jnp.float32)
    off = lax.dot_general(pc, u8, (((1,), (0,)), ((), ())),
                          precision=lax.Precision.HIGHEST,
                          preferred_element_type=jnp.float32)

    pos1 = jnp.sum(one1 * (rank1 + off), axis=1, keepdims=True)
    pos2 = jnp.sum(one2 * (rank2 + off), axis=1, keepdims=True)
    pos_ref[...] = jnp.concatenate([pos1, pos2], axis=1).astype(jnp.int32)
    wts_ref[...] = jnp.concatenate([m1, m2], axis=1)

    blk_start = (off * (1.0 / BM)).astype(jnp.int32)
    bi = lax.broadcasted_iota(jnp.int32, (NBLK, E), 0)
    bmap_ref[...] = jnp.sum((bi >= blk_start).astype(jnp.int32), axis=1,
                            keepdims=True) - 1

    zg = lax.dot_general(x, ws1_ref[...], dn,
                         preferred_element_type=jnp.float32) + bs1_ref[...]
    zu = lax.dot_general(x, ws3_ref[...], dn,
                         preferred_element_type=jnp.float32) + bs3_ref[...]
    z_ref[...] = lax.dot_general(_silu(zg) * zu, ws2_ref[...], dn,
                                 preferred_element_type=jnp.float32) \
        + bs2_ref[...]


def _gate(xf, Wg, bg, Ws1, bs1, Ws3, bs3, Ws2, bs2, interpret=False):
    full = lambda shape: pl.BlockSpec(shape, lambda: tuple(0 for _ in shape))
    return pl.pallas_call(
        _gate_body,
        in_specs=[full((T, DIM)), full((E, DIM)), full((1, E)),
                  full((INTER, DIM)), full((1, INTER)),
                  full((INTER, DIM)), full((1, INTER)),
                  full((DIM, INTER)), full((1, DIM))],
        out_specs=[full((T, 2)), full((T, 2)), full((NBLK, 1)),
                   full((T, DIM))],
        out_shape=[jax.ShapeDtypeStruct((T, 2), jnp.int32),
                   jax.ShapeDtypeStruct((T, 2), jnp.float32),
                   jax.ShapeDtypeStruct((NBLK, 1), jnp.int32),
                   jax.ShapeDtypeStruct((T, DIM), jnp.float32)],
        interpret=interpret,
    )(xf, Wg, bg, Ws1, bs1, Ws3, bs3, Ws2, bs2)


def _dispatch_body(xf_hbm, pos_hbm, wrep_hbm, disp_hbm, w16_hbm,
                   idx_v, rows_v, wv, sem):
    wid = lax.axis_index("s") * 2 + lax.axis_index("c")
    base = wid * TPW
    pltpu.sync_copy(pos_hbm.at[wid], idx_v)
    pltpu.sync_copy(xf_hbm.at[pl.ds(base, TPW)], rows_v)
    pltpu.sync_copy(wrep_hbm.at[wid], wv)
    cps = []
    for j in range(4):
        c = j % 2
        cps.append(pltpu.async_copy(
            rows_v.at[pl.ds(c * 32, 32)], disp_hbm.at[idx_v.at[j]], sem))
        cps.append(pltpu.async_copy(
            wv.at[j], w16_hbm.at[idx_v.at[j]], sem))
    for cp in cps:
        cp.wait()


def _make_dispatch():
    mesh = plsc.VectorSubcoreMesh(core_axis_name="c", subcore_axis_name="s")
    return functools.partial(
        pl.kernel,
        out_type=[jax.ShapeDtypeStruct((P, DIM), jnp.float32),
                  jax.ShapeDtypeStruct((P, 128), jnp.float32)],
        mesh=mesh,
        scratch_types=[pltpu.VMEM((4, 32), jnp.int32),
                       pltpu.VMEM((TPW, DIM), jnp.float32),
                       pltpu.VMEM((4, 32, 128), jnp.float32),
                       pltpu.SemaphoreType.DMA],
    )(_dispatch_body)


def _mlp_body(m_ref, disp_ref, w1_ref, w3_ref, w2_ref, b1_ref, b3_ref,
              b2_ref, w16_ref, out_ref):
    dn = (((1,), (1,)), ((), ()))
    x = disp_ref[...]
    g = lax.dot_general(x, w1_ref[0], dn,
                        preferred_element_type=jnp.float32) + b1_ref[0]
    u = lax.dot_general(x, w3_ref[0], dn,
                        preferred_element_type=jnp.float32) + b3_ref[0]
    h = _silu(g) * u
    out_ref[...] = (lax.dot_general(h, w2_ref[0], dn,
                                    preferred_element_type=jnp.float32)
                    + b2_ref[0]) * w16_ref[:, 0:1]


def _mlp(bmap, disp, W1, W3, W2, b1r, b3r, b2r, w16):
    grid_spec = pltpu.PrefetchScalarGridSpec(
        num_scalar_prefetch=1,
        grid=(NBLK,),
        in_specs=[
            pl.BlockSpec((BM, DIM), lambda b, m: (b, 0)),
            pl.BlockSpec((1, INTER, DIM), lambda b, m: (m[b], 0, 0)),
            pl.BlockSpec((1, INTER, DIM), lambda b, m: (m[b], 0, 0)),
            pl.BlockSpec((1, DIM, INTER), lambda b, m: (m[b], 0, 0)),
            pl.BlockSpec((1, 1, INTER), lambda b, m: (m[b], 0, 0)),
            pl.BlockSpec((1, 1, INTER), lambda b, m: (m[b], 0, 0)),
            pl.BlockSpec((1, 1, DIM), lambda b, m: (m[b], 0, 0)),
            pl.BlockSpec((BM, 128), lambda b, m: (b, 0)),
        ],
        out_specs=pl.BlockSpec((BM, DIM), lambda b, m: (b, 0)),
    )
    return pl.pallas_call(
        _mlp_body,
        grid_spec=grid_spec,
        out_shape=jax.ShapeDtypeStruct((P, DIM), jnp.float32),
    )(bmap, disp, W1, W3, W2, b1r, b3r, b2r, w16)


def _combine_body(out_hbm, pos_hbm, z_hbm, y_hbm,
                  idx_v, r0_v, r1_v, z_v, sem):
    wid = lax.axis_index("s") * 2 + lax.axis_index("c")
    base = wid * TPW
    pltpu.sync_copy(pos_hbm.at[wid], idx_v)

    for c in range(NCHK):
        g0 = pltpu.async_copy(out_hbm.at[idx_v.at[c]], r0_v, sem)
        g1 = pltpu.async_copy(out_hbm.at[idx_v.at[NCHK + c]], r1_v, sem)
        gz = pltpu.async_copy(z_hbm.at[pl.ds(base + c * CT, CT)], z_v, sem)
        g0.wait()
        g1.wait()
        gz.wait()

        def tok(i, _):
            def dchunk(jj, _):
                sl = pl.ds(jj * 16, 16)
                z_v[i, sl] = r0_v[i, sl] + r1_v[i, sl] + z_v[i, sl]
                return 0

            lax.fori_loop(0, DIM // 16, dchunk, 0, unroll=8)
            return 0

        lax.fori_loop(0, CT, tok, 0)
        pltpu.sync_copy(z_v, y_hbm.at[pl.ds(base + c * CT, CT)])


def _make_combine():
    mesh = plsc.VectorSubcoreMesh(core_axis_name="c", subcore_axis_name="s")
    return functools.partial(
        pl.kernel,
        out_type=jax.ShapeDtypeStruct((T, DIM), jnp.float32),
        mesh=mesh,
        scratch_types=[pltpu.VMEM((2 * NCHK, CT), jnp.int32),
                       pltpu.VMEM((CT, DIM), jnp.float32),
                       pltpu.VMEM((CT, DIM), jnp.float32),
                       pltpu.VMEM((CT, DIM), jnp.float32),
                       pltpu.SemaphoreType.DMA],
    )(_combine_body)


@jax.jit
def _moe(xf, Wg, bg, W1, b1r, W3, b3r, W2, b2r, Ws1, bs1, Ws3, bs3, Ws2, bs2):
    pos, wts, bmap, z = _gate(xf, Wg, bg, Ws1, bs1, Ws3, bs3, Ws2, bs2)
    posw = pos.reshape(NW, TPW, 2).transpose(0, 2, 1)
    pos4 = posw.reshape(NW, 2, 2, 32).reshape(NW, 4, 32)
    pos8 = posw.reshape(NW, 2, NCHK, CT).reshape(NW, 2 * NCHK, CT)
    wts4 = wts.reshape(NW, TPW, 2).transpose(0, 2, 1) \
        .reshape(NW, 2, 2, 32).reshape(NW, 4, 32)
    wrep = jnp.broadcast_to(wts4[..., None], (NW, 4, 32, 128))
    disp, w16 = _make_dispatch()(xf, pos4, wrep)
    out = _mlp(bmap.reshape(NBLK), disp, W1, W3, W2, b1r, b3r, b2r, w16)
    y = _make_combine()(out, pos8, z)
    return y


def kernel(x, Wg, bg, W1, b1, W3, b3, W2, b2, Ws1, bs1, Ws3, bs3, Ws2, bs2):
    shape = x.shape
    xf = x.reshape(-1, DIM)
    out = _moe(xf, Wg, bg.reshape(1, E), W1, b1.reshape(E, 1, INTER),
               W3, b3.reshape(E, 1, INTER), W2, b2.reshape(E, 1, DIM),
               Ws1, bs1.reshape(1, INTER), Ws3, bs3.reshape(1, INTER),
               Ws2, bs2.reshape(1, DIM))
    return out.reshape(shape)


# speedup vs baseline: 1.1276x; 1.1276x over previous
import functools

import jax
import jax.numpy as jnp
from jax import lax
from jax.experimental import pallas as pl
from jax.experimental.pallas import tpu as pltpu
from jax.experimental.pallas import tpu_sc as plsc

DIM = 1024
INTER = 512
E = 8
T = 2048
BM = 256
P = 4096 + E * BM
NBLK = P // BM
NW = 32
TPW = T // NW
CT = 32
NCHK = TPW // CT
TB = 512


def _silu(g):
    return g * (1.0 / (1.0 + jnp.exp(-g)))


def _gate_body(x_ref, wg_ref, bg_ref, ws1_ref, bs1_ref, ws3_ref, bs3_ref,
               ws2_ref, bs2_ref, pos_ref, wts_ref, bmap_ref, z_ref):
    dn = (((1,), (1,)), ((), ()))
    x = x_ref[...]
    scores = lax.dot_general(x, wg_ref[...], dn,
                             preferred_element_type=jnp.float32)
    scores = scores + bg_ref[...]
    smax = jnp.max(scores, axis=1, keepdims=True)
    ex = jnp.exp(scores - smax)
    p = ex / jnp.sum(ex, axis=1, keepdims=True)
    iota8 = lax.broadcasted_iota(jnp.int32, (T, E), 1)
    m1 = jnp.max(p, axis=1, keepdims=True)
    i1 = jnp.min(jnp.where(p == m1, iota8, E), axis=1, keepdims=True)
    pm = jnp.where(iota8 == i1, -jnp.inf, p)
    m2 = jnp.max(pm, axis=1, keepdims=True)
    i2 = jnp.min(jnp.where(pm == m2, iota8, E), axis=1, keepdims=True)

    one1 = (iota8 == i1).astype(jnp.float32)
    one2 = (iota8 == i2).astype(jnp.float32)

    r_i = lax.broadcasted_iota(jnp.int32, (T, T), 0)
    c_i = lax.broadcasted_iota(jnp.int32, (T, T), 1)
    ls = (c_i < r_i).astype(jnp.float32)
    rank1 = lax.dot_general(ls, one1, (((1,), (0,)), ((), ())),
                            preferred_element_type=jnp.float32)
    cnt1 = jnp.sum(one1, axis=0, keepdims=True)
    rank2 = lax.dot_general(ls, one2, (((1,), (0,)), ((), ())),
                            preferred_element_type=jnp.float32) + cnt1
    counts = cnt1 + jnp.sum(one2, axis=0, keepdims=True)

    pc = jnp.floor((counts + (BM - 1)) * (1.0 / BM)).astype(jnp.float32) * BM
    rj = lax.broadcasted_iota(jnp.int32, (E, E), 0)
    ce = lax.broadcasted_iota(jnp.int32, (E, E), 1)
    u8 = (rj < ce).astype(jnp.float32)
    off = lax.dot_general(pc, u8, (((1,), (0,)), ((), ())),
                          precision=lax.Precision.HIGHEST,
                          preferred_element_type=jnp.float32)

    pos1 = jnp.sum(one1 * (rank1 + off), axis=1, keepdims=True)
    pos2 = jnp.sum(one2 * (rank2 + off), axis=1, keepdims=True)
    pos_ref[...] = jnp.concatenate([pos1, pos2], axis=1).astype(jnp.int32)
    wts_ref[...] = jnp.concatenate([m1, m2], axis=1)

    blk_start = (off * (1.0 / BM)).astype(jnp.int32)
    bi = lax.broadcasted_iota(jnp.int32, (NBLK, E), 0)
    bmap_ref[...] = jnp.sum((bi >= blk_start).astype(jnp.int32), axis=1,
                            keepdims=True) - 1

    zg = lax.dot_general(x, ws1_ref[...], dn,
                         preferred_element_type=jnp.float32) + bs1_ref[...]
    zu = lax.dot_general(x, ws3_ref[...], dn,
                         preferred_element_type=jnp.float32) + bs3_ref[...]
    z_ref[...] = lax.dot_general(_silu(zg) * zu, ws2_ref[...], dn,
                                 preferred_element_type=jnp.float32) \
        + bs2_ref[...]


def _gate(xf, Wg, bg, Ws1, bs1, Ws3, bs3, Ws2, bs2, interpret=False):
    full = lambda shape: pl.BlockSpec(shape, lambda: tuple(0 for _ in shape))
    return pl.pallas_call(
        _gate_body,
        in_specs=[full((T, DIM)), full((E, DIM)), full((1, E)),
                  full((INTER, DIM)), full((1, INTER)),
                  full((INTER, DIM)), full((1, INTER)),
                  full((DIM, INTER)), full((1, DIM))],
        out_specs=[full((T, 2)), full((T, 2)), full((NBLK, 1)),
                   full((T, DIM))],
        out_shape=[jax.ShapeDtypeStruct((T, 2), jnp.int32),
                   jax.ShapeDtypeStruct((T, 2), jnp.float32),
                   jax.ShapeDtypeStruct((NBLK, 1), jnp.int32),
                   jax.ShapeDtypeStruct((T, DIM), jnp.float32)],
        interpret=interpret,
    )(xf, Wg, bg, Ws1, bs1, Ws3, bs3, Ws2, bs2)


def _dispatch_body(xf_hbm, pos_hbm, wrep_hbm, disp_hbm, w16_hbm,
                   idx_v, rows_v, wv, sem):
    wid = lax.axis_index("s") * 2 + lax.axis_index("c")
    base = wid * TPW
    pltpu.sync_copy(pos_hbm.at[wid], idx_v)
    pltpu.sync_copy(xf_hbm.at[pl.ds(base, TPW)], rows_v)
    pltpu.sync_copy(wrep_hbm.at[wid], wv)
    cps = []
    for j in range(4):
        c = j % 2
        cps.append(pltpu.async_copy(
            rows_v.at[pl.ds(c * 32, 32)], disp_hbm.at[idx_v.at[j]], sem))
        cps.append(pltpu.async_copy(
            wv.at[j], w16_hbm.at[idx_v.at[j]], sem))
    for cp in cps:
        cp.wait()


def _make_dispatch():
    mesh = plsc.VectorSubcoreMesh(core_axis_name="c", subcore_axis_name="s")
    return functools.partial(
        pl.kernel,
        out_type=[jax.ShapeDtypeStruct((P, DIM), jnp.float32),
                  jax.ShapeDtypeStruct((P, 128), jnp.float32)],
        mesh=mesh,
        scratch_types=[pltpu.VMEM((4, 32), jnp.int32),
                       pltpu.VMEM((TPW, DIM), jnp.float32),
                       pltpu.VMEM((4, 32, 128), jnp.float32),
                       pltpu.SemaphoreType.DMA],
    )(_dispatch_body)


def _mlp_body(m_ref, disp_ref, w1_ref, w3_ref, w2_ref, b1_ref, b3_ref,
              b2_ref, w16_ref, out_ref):
    dn = (((1,), (1,)), ((), ()))
    x = disp_ref[...]
    g = lax.dot_general(x, w1_ref[0], dn,
                        preferred_element_type=jnp.float32) + b1_ref[0]
    u = lax.dot_general(x, w3_ref[0], dn,
                        preferred_element_type=jnp.float32) + b3_ref[0]
    h = _silu(g) * u
    out_ref[...] = (lax.dot_general(h, w2_ref[0], dn,
                                    preferred_element_type=jnp.float32)
                    + b2_ref[0]) * w16_ref[:, 0:1]


def _mlp(bmap, disp, W1, W3, W2, b1r, b3r, b2r, w16):
    grid_spec = pltpu.PrefetchScalarGridSpec(
        num_scalar_prefetch=1,
        grid=(NBLK,),
        in_specs=[
            pl.BlockSpec((BM, DIM), lambda b, m: (b, 0)),
            pl.BlockSpec((1, INTER, DIM), lambda b, m: (m[b], 0, 0)),
            pl.BlockSpec((1, INTER, DIM), lambda b, m: (m[b], 0, 0)),
            pl.BlockSpec((1, DIM, INTER), lambda b, m: (m[b], 0, 0)),
            pl.BlockSpec((1, 1, INTER), lambda b, m: (m[b], 0, 0)),
            pl.BlockSpec((1, 1, INTER), lambda b, m: (m[b], 0, 0)),
            pl.BlockSpec((1, 1, DIM), lambda b, m: (m[b], 0, 0)),
            pl.BlockSpec((BM, 128), lambda b, m: (b, 0)),
        ],
        out_specs=pl.BlockSpec((BM, DIM), lambda b, m: (b, 0)),
    )
    return pl.pallas_call(
        _mlp_body,
        grid_spec=grid_spec,
        out_shape=jax.ShapeDtypeStruct((P, DIM), jnp.float32),
    )(bmap, disp, W1, W3, W2, b1r, b3r, b2r, w16)


def _combine_body(out_hbm, pos_hbm, z_hbm, y_hbm,
                  idx_v, r0_v, r1_v, z_v, sem):
    wid = lax.axis_index("s") * 2 + lax.axis_index("c")
    base = wid * TPW
    pltpu.sync_copy(pos_hbm.at[wid], idx_v)

    for c in range(NCHK):
        g0 = pltpu.async_copy(out_hbm.at[idx_v.at[c]], r0_v, sem)
        g1 = pltpu.async_copy(out_hbm.at[idx_v.at[NCHK + c]], r1_v, sem)
        gz = pltpu.async_copy(z_hbm.at[pl.ds(base + c * CT, CT)], z_v, sem)
        g0.wait()
        g1.wait()
        gz.wait()

        def tok(i, _):
            def dchunk(jj, _):
                sl = pl.ds(jj * 16, 16)
                z_v[i, sl] = r0_v[i, sl] + r1_v[i, sl] + z_v[i, sl]
                return 0

            lax.fori_loop(0, DIM // 16, dchunk, 0, unroll=8)
            return 0

        lax.fori_loop(0, CT, tok, 0)
        pltpu.sync_copy(z_v, y_hbm.at[pl.ds(base + c * CT, CT)])


def _make_combine():
    mesh = plsc.VectorSubcoreMesh(core_axis_name="c", subcore_axis_name="s")
    return functools.partial(
        pl.kernel,
        out_type=jax.ShapeDtypeStruct((T, DIM), jnp.float32),
        mesh=mesh,
        scratch_types=[pltpu.VMEM((2 * NCHK, CT), jnp.int32),
                       pltpu.VMEM((CT, DIM), jnp.float32),
                       pltpu.VMEM((CT, DIM), jnp.float32),
                       pltpu.VMEM((CT, DIM), jnp.float32),
                       pltpu.SemaphoreType.DMA],
    )(_combine_body)


@jax.jit
def _moe(xf, Wg, bg, W1, b1r, W3, b3r, W2, b2r, Ws1, bs1, Ws3, bs3, Ws2, bs2):
    pos, wts, bmap, z = _gate(xf, Wg, bg, Ws1, bs1, Ws3, bs3, Ws2, bs2)
    posw = pos.reshape(NW, TPW, 2).transpose(0, 2, 1)
    pos4 = posw.reshape(NW, 2, 2, 32).reshape(NW, 4, 32)
    pos8 = posw.reshape(NW, 2, NCHK, CT).reshape(NW, 2 * NCHK, CT)
    wts4 = wts.reshape(NW, TPW, 2).transpose(0, 2, 1) \
        .reshape(NW, 2, 2, 32).reshape(NW, 4, 32)
    wrep = jnp.broadcast_to(wts4[..., None], (NW, 4, 32, 128))
    disp, w16 = _make_dispatch()(xf, pos4, wrep)
    out = _mlp(bmap.reshape(NBLK), disp, W1, W3, W2, b1r, b3r, b2r, w16)
    y = _make_combine()(out, pos8, z)
    return y


def kernel(x, Wg, bg, W1, b1, W3, b3, W2, b2, Ws1, bs1, Ws3, bs3, Ws2, bs2):
    shape = x.shape
    xf = x.reshape(-1, DIM)
    out = _moe(xf, Wg, bg.reshape(1, E), W1, b1.reshape(E, 1, INTER),
               W3, b3.reshape(E, 1, INTER), W2, b2.reshape(E, 1, DIM),
               Ws1, bs1.reshape(1, INTER), Ws3, bs3.reshape(1, INTER),
               Ws2, bs2.reshape(1, DIM))
    return out.reshape(shape)


# speedup vs baseline: 1.2757x; 1.1313x over previous
import functools

import jax
import jax.numpy as jnp
from jax import lax
from jax.experimental import pallas as pl
from jax.experimental.pallas import tpu as pltpu
from jax.experimental.pallas import tpu_sc as plsc

DIM = 1024
INTER = 512
E = 8
T = 2048
BM = 256
P = 4096 + E * BM
NBLK = P // BM
NW = 32
TPW = T // NW
CT = 32
NCHK = TPW // CT
TB = 512


def _silu(g):
    return g * (1.0 / (1.0 + jnp.exp(-g)))


def _gate_body(x_ref, wg_ref, bg_ref, ws1_ref, bs1_ref, ws3_ref, bs3_ref,
               ws2_ref, bs2_ref, pos_ref, wts_ref, bmap_ref, z_ref):
    dn = (((1,), (1,)), ((), ()))
    x = x_ref[...]
    scores = lax.dot_general(x, wg_ref[...], dn,
                             preferred_element_type=jnp.float32)
    scores = scores + bg_ref[...]
    smax = jnp.max(scores, axis=1, keepdims=True)
    ex = jnp.exp(scores - smax)
    p = ex / jnp.sum(ex, axis=1, keepdims=True)
    iota8 = lax.broadcasted_iota(jnp.int32, (T, E), 1)
    m1 = jnp.max(p, axis=1, keepdims=True)
    i1 = jnp.min(jnp.where(p == m1, iota8, E), axis=1, keepdims=True)
    pm = jnp.where(iota8 == i1, -jnp.inf, p)
    m2 = jnp.max(pm, axis=1, keepdims=True)
    i2 = jnp.min(jnp.where(pm == m2, iota8, E), axis=1, keepdims=True)

    one1 = (iota8 == i1).astype(jnp.float32)
    one2 = (iota8 == i2).astype(jnp.float32)

    r_i = lax.broadcasted_iota(jnp.int32, (T, T), 0)
    c_i = lax.broadcasted_iota(jnp.int32, (T, T), 1)
    ls = (c_i < r_i).astype(jnp.float32)
    rank1 = lax.dot_general(ls, one1, (((1,), (0,)), ((), ())),
                            preferred_element_type=jnp.float32)
    cnt1 = jnp.sum(one1, axis=0, keepdims=True)
    rank2 = lax.dot_general(ls, one2, (((1,), (0,)), ((), ())),
                            preferred_element_type=jnp.float32) + cnt1
    counts = cnt1 + jnp.sum(one2, axis=0, keepdims=True)

    pc = jnp.floor((counts + (BM - 1)) * (1.0 / BM)).astype(jnp.float32) * BM
    rj = lax.broadcasted_iota(jnp.int32, (E, E), 0)
    ce = lax.broadcasted_iota(jnp.int32, (E, E), 1)
    u8 = (rj < ce).astype(jnp.float32)
    off = lax.dot_general(pc, u8, (((1,), (0,)), ((), ())),
                          precision=lax.Precision.HIGHEST,
                          preferred_element_type=jnp.float32)

    pos1 = jnp.sum(one1 * (rank1 + off), axis=1, keepdims=True)
    pos2 = jnp.sum(one2 * (rank2 + off), axis=1, keepdims=True)
    pos_ref[...] = jnp.concatenate([pos1, pos2], axis=1).astype(jnp.int32)
    wts_ref[...] = jnp.concatenate([m1, m2], axis=1)

    blk_start = (off * (1.0 / BM)).astype(jnp.int32)
    bi = lax.broadcasted_iota(jnp.int32, (NBLK, E), 0)
    bmap_ref[...] = jnp.sum((bi >= blk_start).astype(jnp.int32), axis=1,
                            keepdims=True) - 1

    zg = lax.dot_general(x, ws1_ref[...], dn,
                         preferred_element_type=jnp.float32) + bs1_ref[...]
    zu = lax.dot_general(x, ws3_ref[...], dn,
                         preferred_element_type=jnp.float32) + bs3_ref[...]
    z_ref[...] = lax.dot_general(_silu(zg) * zu, ws2_ref[...], dn,
                                 preferred_element_type=jnp.float32) \
        + bs2_ref[...]


def _gate(xf, Wg, bg, Ws1, bs1, Ws3, bs3, Ws2, bs2, interpret=False):
    full = lambda shape: pl.BlockSpec(shape, lambda: tuple(0 for _ in shape))
    return pl.pallas_call(
        _gate_body,
        in_specs=[full((T, DIM)), full((E, DIM)), full((1, E)),
                  full((INTER, DIM)), full((1, INTER)),
                  full((INTER, DIM)), full((1, INTER)),
                  full((DIM, INTER)), full((1, DIM))],
        out_specs=[full((T, 2)), full((T, 2)), full((NBLK, 1)),
                   full((T, DIM))],
        out_shape=[jax.ShapeDtypeStruct((T, 2), jnp.int32),
                   jax.ShapeDtypeStruct((T, 2), jnp.float32),
                   jax.ShapeDtypeStruct((NBLK, 1), jnp.int32),
                   jax.ShapeDtypeStruct((T, DIM), jnp.float32)],
        interpret=interpret,
    )(xf, Wg, bg, Ws1, bs1, Ws3, bs3, Ws2, bs2)


def _dispatch_body(xf_hbm, pos_hbm, wrep_hbm, disp_hbm, w16_hbm,
                   idx_v, rows_v, wv, sem):
    wid = lax.axis_index("s") * 2 + lax.axis_index("c")
    base = wid * TPW
    pltpu.sync_copy(pos_hbm.at[wid], idx_v)
    pltpu.sync_copy(xf_hbm.at[pl.ds(base, TPW)], rows_v)
    pltpu.sync_copy(wrep_hbm.at[wid], wv)
    cps = []
    for j in range(4):
        c = j % 2
        cps.append(pltpu.async_copy(
            rows_v.at[pl.ds(c * 32, 32)], disp_hbm.at[idx_v.at[j]], sem))
        cps.append(pltpu.async_copy(
            wv.at[j], w16_hbm.at[idx_v.at[j]], sem))
    for cp in cps:
        cp.wait()


def _make_dispatch():
    mesh = plsc.VectorSubcoreMesh(core_axis_name="c", subcore_axis_name="s")
    return functools.partial(
        pl.kernel,
        out_type=[jax.ShapeDtypeStruct((P, DIM), jnp.float32),
                  jax.ShapeDtypeStruct((P, 128), jnp.float32)],
        mesh=mesh,
        scratch_types=[pltpu.VMEM((4, 32), jnp.int32),
                       pltpu.VMEM((TPW, DIM), jnp.float32),
                       pltpu.VMEM((4, 32, 128), jnp.float32),
                       pltpu.SemaphoreType.DMA],
    )(_dispatch_body)


def _mlp_body(m_ref, disp_ref, w1_ref, w3_ref, w2_ref, b1_ref, b3_ref,
              b2_ref, w16_ref, out_ref):
    dn = (((1,), (1,)), ((), ()))
    x = disp_ref[...]
    g = lax.dot_general(x, w1_ref[0], dn,
                        preferred_element_type=jnp.float32) + b1_ref[0]
    u = lax.dot_general(x, w3_ref[0], dn,
                        preferred_element_type=jnp.float32) + b3_ref[0]
    h = _silu(g) * u
    out_ref[...] = (lax.dot_general(h, w2_ref[0], dn,
                                    preferred_element_type=jnp.float32)
                    + b2_ref[0]) * w16_ref[:, 0:1]


def _mlp(bmap, disp, W1, W3, W2, b1r, b3r, b2r, w16):
    grid_spec = pltpu.PrefetchScalarGridSpec(
        num_scalar_prefetch=1,
        grid=(NBLK,),
        in_specs=[
            pl.BlockSpec((BM, DIM), lambda b, m: (b, 0)),
            pl.BlockSpec((1, INTER, DIM), lambda b, m: (m[b], 0, 0)),
            pl.BlockSpec((1, INTER, DIM), lambda b, m: (m[b], 0, 0)),
            pl.BlockSpec((1, DIM, INTER), lambda b, m: (m[b], 0, 0)),
            pl.BlockSpec((1, 1, INTER), lambda b, m: (m[b], 0, 0)),
            pl.BlockSpec((1, 1, INTER), lambda b, m: (m[b], 0, 0)),
            pl.BlockSpec((1, 1, DIM), lambda b, m: (m[b], 0, 0)),
            pl.BlockSpec((BM, 128), lambda b, m: (b, 0)),
        ],
        out_specs=pl.BlockSpec((BM, DIM), lambda b, m: (b, 0)),
    )
    return pl.pallas_call(
        _mlp_body,
        grid_spec=grid_spec,
        out_shape=jax.ShapeDtypeStruct((P, DIM), jnp.float32),
    )(bmap, disp, W1, W3, W2, b1r, b3r, b2r, w16)


def _combine_body(out_hbm, pos_hbm, z_hbm, y_hbm,
                  idx_v, r0_v, r1_v, z_v, sem):
    wid = lax.axis_index("s") * 2 + lax.axis_index("c")
    base = wid * TPW
    pltpu.sync_copy(pos_hbm.at[wid], idx_v)

    for c in range(NCHK):
        g0 = pltpu.async_copy(out_hbm.at[idx_v.at[c]], r0_v, sem)
        g1 = pltpu.async_copy(out_hbm.at[idx_v.at[NCHK + c]], r1_v, sem)
        gz = pltpu.async_copy(z_hbm.at[pl.ds(base + c * CT, CT)], z_v, sem)
        g0.wait()
        g1.wait()
        gz.wait()

        def tok(i, _):
            def dchunk(jj, _):
                sl = pl.ds(jj * 16, 16)
                z_v[i, sl] = r0_v[i, sl] + r1_v[i, sl] + z_v[i, sl]
                return 0

            lax.fori_loop(0, DIM // 16, dchunk, 0, unroll=4)
            return 0

        lax.fori_loop(0, CT, tok, 0)
        pltpu.sync_copy(z_v, y_hbm.at[pl.ds(base + c * CT, CT)])


def _make_combine():
    mesh = plsc.VectorSubcoreMesh(core_axis_name="c", subcore_axis_name="s")
    return functools.partial(
        pl.kernel,
        out_type=jax.ShapeDtypeStruct((T, DIM), jnp.float32),
        mesh=mesh,
        scratch_types=[pltpu.VMEM((2 * NCHK, CT), jnp.int32),
                       pltpu.VMEM((CT, DIM), jnp.float32),
                       pltpu.VMEM((CT, DIM), jnp.float32),
                       pltpu.VMEM((CT, DIM), jnp.float32),
                       pltpu.SemaphoreType.DMA],
    )(_combine_body)


@jax.jit
def _moe(xf, Wg, bg, W1, b1r, W3, b3r, W2, b2r, Ws1, bs1, Ws3, bs3, Ws2, bs2):
    pos, wts, bmap, z = _gate(xf, Wg, bg, Ws1, bs1, Ws3, bs3, Ws2, bs2)
    posw = pos.reshape(NW, TPW, 2).transpose(0, 2, 1)
    pos4 = posw.reshape(NW, 2, 2, 32).reshape(NW, 4, 32)
    pos8 = posw.reshape(NW, 2, NCHK, CT).reshape(NW, 2 * NCHK, CT)
    wts4 = wts.reshape(NW, TPW, 2).transpose(0, 2, 1) \
        .reshape(NW, 2, 2, 32).reshape(NW, 4, 32)
    wrep = jnp.broadcast_to(wts4[..., None], (NW, 4, 32, 128))
    disp, w16 = _make_dispatch()(xf, pos4, wrep)
    out = _mlp(bmap.reshape(NBLK), disp, W1, W3, W2, b1r, b3r, b2r, w16)
    y = _make_combine()(out, pos8, z)
    return y


def kernel(x, Wg, bg, W1, b1, W3, b3, W2, b2, Ws1, bs1, Ws3, bs3, Ws2, bs2):
    shape = x.shape
    xf = x.reshape(-1, DIM)
    out = _moe(xf, Wg, bg.reshape(1, E), W1, b1.reshape(E, 1, INTER),
               W3, b3.reshape(E, 1, INTER), W2, b2.reshape(E, 1, DIM),
               Ws1, bs1.reshape(1, INTER), Ws3, bs3.reshape(1, INTER),
               Ws2, bs2.reshape(1, DIM))
    return out.reshape(shape)


# speedup vs baseline: 1.3216x; 1.0360x over previous
import functools

import jax
import jax.numpy as jnp
from jax import lax
from jax.experimental import pallas as pl
from jax.experimental.pallas import tpu as pltpu
from jax.experimental.pallas import tpu_sc as plsc

DIM = 1024
INTER = 512
E = 8
T = 2048
BM = 256
P = 4096 + E * BM
NBLK = P // BM
NW = 32
TPW = T // NW
CT = 32
NCHK = TPW // CT
TB = 512


def _silu(g):
    return g * (1.0 / (1.0 + jnp.exp(-g)))


def _gate_body(x_ref, wg_ref, bg_ref, pos_ref, wts_ref, bmap_ref):
    dn = (((1,), (1,)), ((), ()))
    x = x_ref[...]
    scores = lax.dot_general(x, wg_ref[...], dn,
                             preferred_element_type=jnp.float32)
    scores = scores + bg_ref[...]
    smax = jnp.max(scores, axis=1, keepdims=True)
    ex = jnp.exp(scores - smax)
    p = ex / jnp.sum(ex, axis=1, keepdims=True)
    iota8 = lax.broadcasted_iota(jnp.int32, (T, E), 1)
    m1 = jnp.max(p, axis=1, keepdims=True)
    i1 = jnp.min(jnp.where(p == m1, iota8, E), axis=1, keepdims=True)
    pm = jnp.where(iota8 == i1, -jnp.inf, p)
    m2 = jnp.max(pm, axis=1, keepdims=True)
    i2 = jnp.min(jnp.where(pm == m2, iota8, E), axis=1, keepdims=True)

    one1 = (iota8 == i1).astype(jnp.float32)
    one2 = (iota8 == i2).astype(jnp.float32)

    r_i = lax.broadcasted_iota(jnp.int32, (T, T), 0)
    c_i = lax.broadcasted_iota(jnp.int32, (T, T), 1)
    ls = (c_i < r_i).astype(jnp.float32)
    rank1 = lax.dot_general(ls, one1, (((1,), (0,)), ((), ())),
                            preferred_element_type=jnp.float32)
    cnt1 = jnp.sum(one1, axis=0, keepdims=True)
    rank2 = lax.dot_general(ls, one2, (((1,), (0,)), ((), ())),
                            preferred_element_type=jnp.float32) + cnt1
    counts = cnt1 + jnp.sum(one2, axis=0, keepdims=True)

    pc = jnp.floor((counts + (BM - 1)) * (1.0 / BM)).astype(jnp.float32) * BM
    rj = lax.broadcasted_iota(jnp.int32, (E, E), 0)
    ce = lax.broadcasted_iota(jnp.int32, (E, E), 1)
    u8 = (rj < ce).astype(jnp.float32)
    off = lax.dot_general(pc, u8, (((1,), (0,)), ((), ())),
                          precision=lax.Precision.HIGHEST,
                          preferred_element_type=jnp.float32)

    pos1 = jnp.sum(one1 * (rank1 + off), axis=1, keepdims=True)
    pos2 = jnp.sum(one2 * (rank2 + off), axis=1, keepdims=True)
    pos_ref[...] = jnp.concatenate([pos1, pos2], axis=1).astype(jnp.int32)
    wts_ref[...] = jnp.concatenate([m1, m2], axis=1)

    blk_start = (off * (1.0 / BM)).astype(jnp.int32)
    bi = lax.broadcasted_iota(jnp.int32, (NBLK, E), 0)
    bmap_ref[...] = jnp.sum((bi >= blk_start).astype(jnp.int32), axis=1,
                            keepdims=True) - 1


def _gate(xf, Wg, bg, interpret=False):
    full = lambda shape: pl.BlockSpec(shape, lambda: tuple(0 for _ in shape))
    return pl.pallas_call(
        _gate_body,
        in_specs=[full((T, DIM)), full((E, DIM)), full((1, E))],
        out_specs=[full((T, 2)), full((T, 2)), full((NBLK, 1))],
        out_shape=[jax.ShapeDtypeStruct((T, 2), jnp.int32),
                   jax.ShapeDtypeStruct((T, 2), jnp.float32),
                   jax.ShapeDtypeStruct((NBLK, 1), jnp.int32)],
        interpret=interpret,
    )(xf, Wg, bg)


def _shared_body(x_ref, ws1_ref, bs1_ref, ws3_ref, bs3_ref, ws2_ref, bs2_ref,
                 z_ref):
    dn = (((1,), (1,)), ((), ()))
    x = x_ref[...]
    zg = lax.dot_general(x, ws1_ref[...], dn,
                         preferred_element_type=jnp.float32) + bs1_ref[...]
    zu = lax.dot_general(x, ws3_ref[...], dn,
                         preferred_element_type=jnp.float32) + bs3_ref[...]
    z_ref[...] = lax.dot_general(_silu(zg) * zu, ws2_ref[...], dn,
                                 preferred_element_type=jnp.float32) \
        + bs2_ref[...]


def _shared(xf, Ws1, bs1, Ws3, bs3, Ws2, bs2):
    c2 = lambda shape: pl.BlockSpec(shape, lambda i: (0, 0))
    return pl.pallas_call(
        _shared_body,
        grid=(T // TB,),
        in_specs=[pl.BlockSpec((TB, DIM), lambda i: (i, 0)),
                  c2((INTER, DIM)), c2((1, INTER)),
                  c2((INTER, DIM)), c2((1, INTER)),
                  c2((DIM, INTER)), c2((1, DIM))],
        out_specs=pl.BlockSpec((TB, DIM), lambda i: (i, 0)),
        out_shape=jax.ShapeDtypeStruct((T, DIM), jnp.float32),
    )(xf, Ws1, bs1, Ws3, bs3, Ws2, bs2)


def _dispatch_body(xf_hbm, pos_hbm, wrep_hbm, disp_hbm, w16_hbm,
                   idx_v, rows_v, wv, sem):
    wid = lax.axis_index("s") * 2 + lax.axis_index("c")
    base = wid * TPW
    pltpu.sync_copy(pos_hbm.at[wid], idx_v)
    pltpu.sync_copy(xf_hbm.at[pl.ds(base, TPW)], rows_v)
    pltpu.sync_copy(wrep_hbm.at[wid], wv)
    cps = []
    for j in range(4):
        c = j % 2
        cps.append(pltpu.async_copy(
            rows_v.at[pl.ds(c * 32, 32)], disp_hbm.at[idx_v.at[j]], sem))
        cps.append(pltpu.async_copy(
            wv.at[j], w16_hbm.at[idx_v.at[j]], sem))
    for cp in cps:
        cp.wait()


def _make_dispatch():
    mesh = plsc.VectorSubcoreMesh(core_axis_name="c", subcore_axis_name="s")
    return functools.partial(
        pl.kernel,
        out_type=[jax.ShapeDtypeStruct((P, DIM), jnp.float32),
                  jax.ShapeDtypeStruct((P, 128), jnp.float32)],
        mesh=mesh,
        scratch_types=[pltpu.VMEM((4, 32), jnp.int32),
                       pltpu.VMEM((TPW, DIM), jnp.float32),
                       pltpu.VMEM((4, 32, 128), jnp.float32),
                       pltpu.SemaphoreType.DMA],
    )(_dispatch_body)


def _mlp_body(m_ref, disp_ref, w1_ref, w3_ref, w2_ref, b1_ref, b3_ref,
              b2_ref, w16_ref, out_ref):
    dn = (((1,), (1,)), ((), ()))
    x = disp_ref[...]
    g = lax.dot_general(x, w1_ref[0], dn,
                        preferred_element_type=jnp.float32) + b1_ref[0]
    u = lax.dot_general(x, w3_ref[0], dn,
                        preferred_element_type=jnp.float32) + b3_ref[0]
    h = _silu(g) * u
    out_ref[...] = (lax.dot_general(h, w2_ref[0], dn,
                                    preferred_element_type=jnp.float32)
                    + b2_ref[0]) * w16_ref[:, 0:1]


def _mlp(bmap, disp, W1, W3, W2, b1r, b3r, b2r, w16):
    grid_spec = pltpu.PrefetchScalarGridSpec(
        num_scalar_prefetch=1,
        grid=(NBLK,),
        in_specs=[
            pl.BlockSpec((BM, DIM), lambda b, m: (b, 0)),
            pl.BlockSpec((1, INTER, DIM), lambda b, m: (m[b], 0, 0)),
            pl.BlockSpec((1, INTER, DIM), lambda b, m: (m[b], 0, 0)),
            pl.BlockSpec((1, DIM, INTER), lambda b, m: (m[b], 0, 0)),
            pl.BlockSpec((1, 1, INTER), lambda b, m: (m[b], 0, 0)),
            pl.BlockSpec((1, 1, INTER), lambda b, m: (m[b], 0, 0)),
            pl.BlockSpec((1, 1, DIM), lambda b, m: (m[b], 0, 0)),
            pl.BlockSpec((BM, 128), lambda b, m: (b, 0)),
        ],
        out_specs=pl.BlockSpec((BM, DIM), lambda b, m: (b, 0)),
    )
    return pl.pallas_call(
        _mlp_body,
        grid_spec=grid_spec,
        out_shape=jax.ShapeDtypeStruct((P, DIM), jnp.float32),
    )(bmap, disp, W1, W3, W2, b1r, b3r, b2r, w16)


def _combine_body(out_hbm, pos_hbm, z_hbm, y_hbm,
                  idx_v, r0_v, r1_v, z_v, sem):
    wid = lax.axis_index("s") * 2 + lax.axis_index("c")
    base = wid * TPW
    pltpu.sync_copy(pos_hbm.at[wid], idx_v)

    for c in range(NCHK):
        g0 = pltpu.async_copy(out_hbm.at[idx_v.at[c]], r0_v, sem)
        g1 = pltpu.async_copy(out_hbm.at[idx_v.at[NCHK + c]], r1_v, sem)
        gz = pltpu.async_copy(z_hbm.at[pl.ds(base + c * CT, CT)], z_v, sem)
        g0.wait()
        g1.wait()
        gz.wait()

        def tok(i, _):
            def dchunk(jj, _):
                sl = pl.ds(jj * 16, 16)
                z_v[i, sl] = r0_v[i, sl] + r1_v[i, sl] + z_v[i, sl]
                return 0

            lax.fori_loop(0, DIM // 16, dchunk, 0, unroll=4)
            return 0

        lax.fori_loop(0, CT, tok, 0)
        pltpu.sync_copy(z_v, y_hbm.at[pl.ds(base + c * CT, CT)])


def _make_combine():
    mesh = plsc.VectorSubcoreMesh(core_axis_name="c", subcore_axis_name="s")
    return functools.partial(
        pl.kernel,
        out_type=jax.ShapeDtypeStruct((T, DIM), jnp.float32),
        mesh=mesh,
        scratch_types=[pltpu.VMEM((2 * NCHK, CT), jnp.int32),
                       pltpu.VMEM((CT, DIM), jnp.float32),
                       pltpu.VMEM((CT, DIM), jnp.float32),
                       pltpu.VMEM((CT, DIM), jnp.float32),
                       pltpu.SemaphoreType.DMA],
    )(_combine_body)


@jax.jit
def _moe(xf, Wg, bg, W1, b1r, W3, b3r, W2, b2r, Ws1, bs1, Ws3, bs3, Ws2, bs2):
    pos, wts, bmap = _gate(xf, Wg, bg)
    pos4 = pos.reshape(NW, TPW, 2).transpose(0, 2, 1) \
        .reshape(NW, 2, 2, 32).reshape(NW, 4, 32)
    wrep = jnp.broadcast_to(
        wts.reshape(NW, TPW, 2).transpose(0, 2, 1)
        .reshape(NW, 2, 2, 32).reshape(NW, 4, 32)[..., None],
        (NW, 4, 32, 128))
    disp, w16 = _make_dispatch()(xf, pos4, wrep)
    z = _shared(xf, Ws1, bs1, Ws3, bs3, Ws2, bs2)
    out = _mlp(bmap.reshape(NBLK), disp, W1, W3, W2, b1r, b3r, b2r, w16)
    y = _make_combine()(out, pos4, z)
    return y


def kernel(x, Wg, bg, W1, b1, W3, b3, W2, b2, Ws1, bs1, Ws3, bs3, Ws2, bs2):
    shape = x.shape
    xf = x.reshape(-1, DIM)
    out = _moe(xf, Wg, bg.reshape(1, E), W1, b1.reshape(E, 1, INTER),
               W3, b3.reshape(E, 1, INTER), W2, b2.reshape(E, 1, DIM),
               Ws1, bs1.reshape(1, INTER), Ws3, bs3.reshape(1, INTER),
               Ws2, bs2.reshape(1, DIM))
    return out.reshape(shape)


# speedup vs baseline: 1.3430x; 1.0162x over previous
import functools

import jax
import jax.numpy as jnp
from jax import lax
from jax.experimental import pallas as pl
from jax.experimental.pallas import tpu as pltpu
from jax.experimental.pallas import tpu_sc as plsc

DIM = 1024
INTER = 512
E = 8
T = 2048
BM = 256
P = 4096 + E * BM
NBLK = P // BM
NW = 32
TPW = T // NW
CT = 32
NCHK = TPW // CT
TB = 512


def _silu(g):
    return g * (1.0 / (1.0 + jnp.exp(-g)))


def _gate_body(x_ref, wg_ref, bg_ref, pos_ref, wts_ref, bmap_ref):
    dn = (((1,), (1,)), ((), ()))
    x = x_ref[...]
    scores = lax.dot_general(x, wg_ref[...], dn,
                             preferred_element_type=jnp.float32)
    scores = scores + bg_ref[...]
    smax = jnp.max(scores, axis=1, keepdims=True)
    ex = jnp.exp(scores - smax)
    p = ex / jnp.sum(ex, axis=1, keepdims=True)
    iota8 = lax.broadcasted_iota(jnp.int32, (T, E), 1)
    m1 = jnp.max(p, axis=1, keepdims=True)
    i1 = jnp.min(jnp.where(p == m1, iota8, E), axis=1, keepdims=True)
    pm = jnp.where(iota8 == i1, -jnp.inf, p)
    m2 = jnp.max(pm, axis=1, keepdims=True)
    i2 = jnp.min(jnp.where(pm == m2, iota8, E), axis=1, keepdims=True)

    one1 = (iota8 == i1).astype(jnp.float32)
    one2 = (iota8 == i2).astype(jnp.float32)

    r_i = lax.broadcasted_iota(jnp.int32, (T, T), 0)
    c_i = lax.broadcasted_iota(jnp.int32, (T, T), 1)
    ls = (c_i < r_i).astype(jnp.float32)
    rank1 = lax.dot_general(ls, one1, (((1,), (0,)), ((), ())),
                            preferred_element_type=jnp.float32)
    cnt1 = jnp.sum(one1, axis=0, keepdims=True)
    rank2 = lax.dot_general(ls, one2, (((1,), (0,)), ((), ())),
                            preferred_element_type=jnp.float32) + cnt1
    counts = cnt1 + jnp.sum(one2, axis=0, keepdims=True)

    pc = jnp.floor((counts + (BM - 1)) * (1.0 / BM)).astype(jnp.float32) * BM
    rj = lax.broadcasted_iota(jnp.int32, (E, E), 0)
    ce = lax.broadcasted_iota(jnp.int32, (E, E), 1)
    u8 = (rj < ce).astype(jnp.float32)
    off = lax.dot_general(pc, u8, (((1,), (0,)), ((), ())),
                          precision=lax.Precision.HIGHEST,
                          preferred_element_type=jnp.float32)

    pos1 = jnp.sum(one1 * (rank1 + off), axis=1, keepdims=True)
    pos2 = jnp.sum(one2 * (rank2 + off), axis=1, keepdims=True)
    pos_ref[...] = jnp.concatenate([pos1, pos2], axis=1).astype(jnp.int32)
    wts_ref[...] = jnp.concatenate([m1, m2], axis=1)

    blk_start = (off * (1.0 / BM)).astype(jnp.int32)
    bi = lax.broadcasted_iota(jnp.int32, (NBLK, E), 0)
    bmap_ref[...] = jnp.sum((bi >= blk_start).astype(jnp.int32), axis=1,
                            keepdims=True) - 1


def _gate(xf, Wg, bg, interpret=False):
    full = lambda shape: pl.BlockSpec(shape, lambda: tuple(0 for _ in shape))
    return pl.pallas_call(
        _gate_body,
        in_specs=[full((T, DIM)), full((E, DIM)), full((1, E))],
        out_specs=[full((T, 2)), full((T, 2)), full((NBLK, 1))],
        out_shape=[jax.ShapeDtypeStruct((T, 2), jnp.int32),
                   jax.ShapeDtypeStruct((T, 2), jnp.float32),
                   jax.ShapeDtypeStruct((NBLK, 1), jnp.int32)],
        interpret=interpret,
    )(xf, Wg, bg)


def _shared_body(x_ref, ws1_ref, bs1_ref, ws3_ref, bs3_ref, ws2_ref, bs2_ref,
                 z_ref):
    dn = (((1,), (1,)), ((), ()))
    x = x_ref[...]
    zg = lax.dot_general(x, ws1_ref[...], dn,
                         preferred_element_type=jnp.float32) + bs1_ref[...]
    zu = lax.dot_general(x, ws3_ref[...], dn,
                         preferred_element_type=jnp.float32) + bs3_ref[...]
    z_ref[...] = lax.dot_general(_silu(zg) * zu, ws2_ref[...], dn,
                                 preferred_element_type=jnp.float32) \
        + bs2_ref[...]


def _shared(xf, Ws1, bs1, Ws3, bs3, Ws2, bs2):
    c2 = lambda shape: pl.BlockSpec(shape, lambda i: (0, 0))
    return pl.pallas_call(
        _shared_body,
        grid=(T // TB,),
        in_specs=[pl.BlockSpec((TB, DIM), lambda i: (i, 0)),
                  c2((INTER, DIM)), c2((1, INTER)),
                  c2((INTER, DIM)), c2((1, INTER)),
                  c2((DIM, INTER)), c2((1, DIM))],
        out_specs=pl.BlockSpec((TB, DIM), lambda i: (i, 0)),
        out_shape=jax.ShapeDtypeStruct((T, DIM), jnp.float32),
    )(xf, Ws1, bs1, Ws3, bs3, Ws2, bs2)


def _dispatch_body(xf_hbm, pos_hbm, wrep_hbm, disp_hbm, w16_hbm,
                   idx_v, rows_v, wv, sem):
    wid = lax.axis_index("s") * 2 + lax.axis_index("c")
    base = wid * TPW
    pltpu.sync_copy(pos_hbm.at[wid], idx_v)
    pltpu.sync_copy(xf_hbm.at[pl.ds(base, TPW)], rows_v)
    pltpu.sync_copy(wrep_hbm.at[wid], wv)
    cps = []
    for j in range(4):
        c = j % 2
        cps.append(pltpu.async_copy(
            rows_v.at[pl.ds(c * 32, 32)], disp_hbm.at[idx_v.at[j]], sem))
        cps.append(pltpu.async_copy(
            wv.at[j], w16_hbm.at[idx_v.at[j]], sem))
    for cp in cps:
        cp.wait()


def _make_dispatch():
    mesh = plsc.VectorSubcoreMesh(core_axis_name="c", subcore_axis_name="s")
    return functools.partial(
        pl.kernel,
        out_type=[jax.ShapeDtypeStruct((P, DIM), jnp.float32),
                  jax.ShapeDtypeStruct((P, 128), jnp.float32)],
        mesh=mesh,
        scratch_types=[pltpu.VMEM((4, 32), jnp.int32),
                       pltpu.VMEM((TPW, DIM), jnp.float32),
                       pltpu.VMEM((4, 32, 128), jnp.float32),
                       pltpu.SemaphoreType.DMA],
    )(_dispatch_body)


def _mlp_body(m_ref, disp_ref, w1_ref, w3_ref, w2_ref, b1_ref, b3_ref,
              b2_ref, w16_ref, out_ref):
    dn = (((1,), (1,)), ((), ()))
    e = m_ref[pl.program_id(0)]
    x = disp_ref[...]
    g = lax.dot_general(x, w1_ref[e], dn,
                        preferred_element_type=jnp.float32) + b1_ref[e]
    u = lax.dot_general(x, w3_ref[e], dn,
                        preferred_element_type=jnp.float32) + b3_ref[e]
    h = _silu(g) * u
    out_ref[...] = (lax.dot_general(h, w2_ref[e], dn,
                                    preferred_element_type=jnp.float32)
                    + b2_ref[e]) * w16_ref[:, 0:1]


def _mlp(bmap, disp, W1, W3, W2, b1r, b3r, b2r, w16):
    grid_spec = pltpu.PrefetchScalarGridSpec(
        num_scalar_prefetch=1,
        grid=(NBLK,),
        in_specs=[
            pl.BlockSpec((BM, DIM), lambda b, m: (b, 0)),
            pl.BlockSpec((E, INTER, DIM), lambda b, m: (0, 0, 0)),
            pl.BlockSpec((E, INTER, DIM), lambda b, m: (0, 0, 0)),
            pl.BlockSpec((E, DIM, INTER), lambda b, m: (0, 0, 0)),
            pl.BlockSpec((E, 1, INTER), lambda b, m: (0, 0, 0)),
            pl.BlockSpec((E, 1, INTER), lambda b, m: (0, 0, 0)),
            pl.BlockSpec((E, 1, DIM), lambda b, m: (0, 0, 0)),
            pl.BlockSpec((BM, 128), lambda b, m: (b, 0)),
        ],
        out_specs=pl.BlockSpec((BM, DIM), lambda b, m: (b, 0)),
    )
    return pl.pallas_call(
        _mlp_body,
        grid_spec=grid_spec,
        out_shape=jax.ShapeDtypeStruct((P, DIM), jnp.float32),
    )(bmap, disp, W1, W3, W2, b1r, b3r, b2r, w16)


def _combine_body(out_hbm, pos_hbm, z_hbm, y_hbm,
                  idx_v, r0_v, r1_v, z_v, sem):
    wid = lax.axis_index("s") * 2 + lax.axis_index("c")
    base = wid * TPW
    pltpu.sync_copy(pos_hbm.at[wid], idx_v)

    for c in range(NCHK):
        g0 = pltpu.async_copy(out_hbm.at[idx_v.at[c]], r0_v, sem)
        g1 = pltpu.async_copy(out_hbm.at[idx_v.at[NCHK + c]], r1_v, sem)
        gz = pltpu.async_copy(z_hbm.at[pl.ds(base + c * CT, CT)], z_v, sem)
        g0.wait()
        g1.wait()
        gz.wait()

        def tok(i, _):
            def dchunk(jj, _):
                sl = pl.ds(jj * 16, 16)
                z_v[i, sl] = r0_v[i, sl] + r1_v[i, sl] + z_v[i, sl]
                return 0

            lax.fori_loop(0, DIM // 16, dchunk, 0, unroll=4)
            return 0

        lax.fori_loop(0, CT, tok, 0)
        pltpu.sync_copy(z_v, y_hbm.at[pl.ds(base + c * CT, CT)])


def _make_combine():
    mesh = plsc.VectorSubcoreMesh(core_axis_name="c", subcore_axis_name="s")
    return functools.partial(
        pl.kernel,
        out_type=jax.ShapeDtypeStruct((T, DIM), jnp.float32),
        mesh=mesh,
        scratch_types=[pltpu.VMEM((2 * NCHK, CT), jnp.int32),
                       pltpu.VMEM((CT, DIM), jnp.float32),
                       pltpu.VMEM((CT, DIM), jnp.float32),
                       pltpu.VMEM((CT, DIM), jnp.float32),
                       pltpu.SemaphoreType.DMA],
    )(_combine_body)


@jax.jit
def _moe(xf, Wg, bg, W1, b1r, W3, b3r, W2, b2r, Ws1, bs1, Ws3, bs3, Ws2, bs2):
    pos, wts, bmap = _gate(xf, Wg, bg)
    pos4 = pos.reshape(NW, TPW, 2).transpose(0, 2, 1) \
        .reshape(NW, 2, 2, 32).reshape(NW, 4, 32)
    wrep = jnp.broadcast_to(
        wts.reshape(NW, TPW, 2).transpose(0, 2, 1)
        .reshape(NW, 2, 2, 32).reshape(NW, 4, 32)[..., None],
        (NW, 4, 32, 128))
    disp, w16 = _make_dispatch()(xf, pos4, wrep)
    z = _shared(xf, Ws1, bs1, Ws3, bs3, Ws2, bs2)
    out = _mlp(bmap.reshape(NBLK), disp, W1, W3, W2, b1r, b3r, b2r, w16)
    y = _make_combine()(out, pos4, z)
    return y


def kernel(x, Wg, bg, W1, b1, W3, b3, W2, b2, Ws1, bs1, Ws3, bs3, Ws2, bs2):
    shape = x.shape
    xf = x.reshape(-1, DIM)
    out = _moe(xf, Wg, bg.reshape(1, E), W1, b1.reshape(E, 1, INTER),
               W3, b3.reshape(E, 1, INTER), W2, b2.reshape(E, 1, DIM),
               Ws1, bs1.reshape(1, INTER), Ws3, bs3.reshape(1, INTER),
               Ws2, bs2.reshape(1, DIM))
    return out.reshape(shape)


# speedup vs baseline: 1.3861x; 1.0321x over previous
import functools

import jax
import jax.numpy as jnp
from jax import lax
from jax.experimental import pallas as pl
from jax.experimental.pallas import tpu as pltpu
from jax.experimental.pallas import tpu_sc as plsc

DIM = 1024
INTER = 512
E = 8
T = 2048
BM = 256
P = 4096 + E * BM
NBLK = P // BM
NW = 32
TPW = T // NW
CT = 32
NCHK = TPW // CT
TB = 512


def _silu(g):
    return g * (1.0 / (1.0 + jnp.exp(-g)))


def _gate_body(x_ref, wg_ref, bg_ref, pos_ref, wts_ref, bmap_ref):
    dn = (((1,), (1,)), ((), ()))
    x = x_ref[...]
    scores = lax.dot_general(x, wg_ref[...], dn,
                             preferred_element_type=jnp.float32)
    scores = scores + bg_ref[...]
    smax = jnp.max(scores, axis=1, keepdims=True)
    ex = jnp.exp(scores - smax)
    p = ex / jnp.sum(ex, axis=1, keepdims=True)
    iota8 = lax.broadcasted_iota(jnp.int32, (T, E), 1)
    m1 = jnp.max(p, axis=1, keepdims=True)
    i1 = jnp.min(jnp.where(p == m1, iota8, E), axis=1, keepdims=True)
    pm = jnp.where(iota8 == i1, -jnp.inf, p)
    m2 = jnp.max(pm, axis=1, keepdims=True)
    i2 = jnp.min(jnp.where(pm == m2, iota8, E), axis=1, keepdims=True)

    one1 = (iota8 == i1).astype(jnp.float32)
    one2 = (iota8 == i2).astype(jnp.float32)

    r_i = lax.broadcasted_iota(jnp.int32, (T, T), 0)
    c_i = lax.broadcasted_iota(jnp.int32, (T, T), 1)
    ls = (c_i < r_i).astype(jnp.float32)
    rank1 = lax.dot_general(ls, one1, (((1,), (0,)), ((), ())),
                            preferred_element_type=jnp.float32)
    cnt1 = jnp.sum(one1, axis=0, keepdims=True)
    rank2 = lax.dot_general(ls, one2, (((1,), (0,)), ((), ())),
                            preferred_element_type=jnp.float32) + cnt1
    counts = cnt1 + jnp.sum(one2, axis=0, keepdims=True)

    pc = jnp.floor((counts + (BM - 1)) * (1.0 / BM)).astype(jnp.float32) * BM
    rj = lax.broadcasted_iota(jnp.int32, (E, E), 0)
    ce = lax.broadcasted_iota(jnp.int32, (E, E), 1)
    u8 = (rj < ce).astype(jnp.float32)
    off = lax.dot_general(pc, u8, (((1,), (0,)), ((), ())),
                          precision=lax.Precision.HIGHEST,
                          preferred_element_type=jnp.float32)

    pos1 = jnp.sum(one1 * (rank1 + off), axis=1, keepdims=True)
    pos2 = jnp.sum(one2 * (rank2 + off), axis=1, keepdims=True)
    pos_ref[...] = jnp.concatenate([pos1, pos2], axis=1).astype(jnp.int32)
    wts_ref[...] = jnp.concatenate([m1, m2], axis=1)

    blk_start = (off * (1.0 / BM)).astype(jnp.int32)
    bi = lax.broadcasted_iota(jnp.int32, (NBLK + 1, E), 0)
    bmap = jnp.sum((bi >= blk_start).astype(jnp.int32), axis=1,
                   keepdims=True) - 1
    used = ((off + pc) * (1.0 / BM)).astype(jnp.int32)
    ub = jnp.max(used, axis=1, keepdims=True)
    ri = lax.broadcasted_iota(jnp.int32, (NBLK + 1, 1), 0)
    bmap_ref[...] = jnp.where(ri == NBLK, ub, bmap)


def _gate(xf, Wg, bg, interpret=False):
    full = lambda shape: pl.BlockSpec(shape, lambda: tuple(0 for _ in shape))
    return pl.pallas_call(
        _gate_body,
        in_specs=[full((T, DIM)), full((E, DIM)), full((1, E))],
        out_specs=[full((T, 2)), full((T, 2)), full((NBLK + 1, 1))],
        out_shape=[jax.ShapeDtypeStruct((T, 2), jnp.int32),
                   jax.ShapeDtypeStruct((T, 2), jnp.float32),
                   jax.ShapeDtypeStruct((NBLK + 1, 1), jnp.int32)],
        interpret=interpret,
    )(xf, Wg, bg)


def _shared_body(x_ref, ws1_ref, bs1_ref, ws3_ref, bs3_ref, ws2_ref, bs2_ref,
                 z_ref):
    dn = (((1,), (1,)), ((), ()))
    x = x_ref[...]
    zg = lax.dot_general(x, ws1_ref[...], dn,
                         preferred_element_type=jnp.float32) + bs1_ref[...]
    zu = lax.dot_general(x, ws3_ref[...], dn,
                         preferred_element_type=jnp.float32) + bs3_ref[...]
    z_ref[...] = lax.dot_general(_silu(zg) * zu, ws2_ref[...], dn,
                                 preferred_element_type=jnp.float32) \
        + bs2_ref[...]


def _shared(xf, Ws1, bs1, Ws3, bs3, Ws2, bs2):
    c2 = lambda shape: pl.BlockSpec(shape, lambda i: (0, 0))
    return pl.pallas_call(
        _shared_body,
        grid=(T // TB,),
        in_specs=[pl.BlockSpec((TB, DIM), lambda i: (i, 0)),
                  c2((INTER, DIM)), c2((1, INTER)),
                  c2((INTER, DIM)), c2((1, INTER)),
                  c2((DIM, INTER)), c2((1, DIM))],
        out_specs=pl.BlockSpec((TB, DIM), lambda i: (i, 0)),
        out_shape=jax.ShapeDtypeStruct((T, DIM), jnp.float32),
    )(xf, Ws1, bs1, Ws3, bs3, Ws2, bs2)


def _dispatch_body(xf_hbm, pos_hbm, wrep_hbm, disp_hbm, w16_hbm,
                   idx_v, rows_v, wv, sem):
    wid = lax.axis_index("s") * 2 + lax.axis_index("c")
    base = wid * TPW
    pltpu.sync_copy(pos_hbm.at[wid], idx_v)
    pltpu.sync_copy(xf_hbm.at[pl.ds(base, TPW)], rows_v)
    pltpu.sync_copy(wrep_hbm.at[wid], wv)
    cps = []
    for j in range(4):
        c = j % 2
        cps.append(pltpu.async_copy(
            rows_v.at[pl.ds(c * 32, 32)], disp_hbm.at[idx_v.at[j]], sem))
        cps.append(pltpu.async_copy(
            wv.at[j], w16_hbm.at[idx_v.at[j]], sem))
    for cp in cps:
        cp.wait()


def _make_dispatch():
    mesh = plsc.VectorSubcoreMesh(core_axis_name="c", subcore_axis_name="s")
    return functools.partial(
        pl.kernel,
        out_type=[jax.ShapeDtypeStruct((P, DIM), jnp.float32),
                  jax.ShapeDtypeStruct((P, 128), jnp.float32)],
        mesh=mesh,
        scratch_types=[pltpu.VMEM((4, 32), jnp.int32),
                       pltpu.VMEM((TPW, DIM), jnp.float32),
                       pltpu.VMEM((4, 32, 128), jnp.float32),
                       pltpu.SemaphoreType.DMA],
    )(_dispatch_body)


def _mlp_body(m_ref, disp_ref, w1_ref, w3_ref, w2_ref, b1_ref, b3_ref,
              b2_ref, w16_ref, out_ref):
    dn = (((1,), (1,)), ((), ()))
    b = pl.program_id(0)
    e = m_ref[b]

    @pl.when(b < m_ref[NBLK])
    def _do():
        x = disp_ref[...]
        g = lax.dot_general(x, w1_ref[e], dn,
                            preferred_element_type=jnp.float32) + b1_ref[e]
        u = lax.dot_general(x, w3_ref[e], dn,
                            preferred_element_type=jnp.float32) + b3_ref[e]
        h = _silu(g) * u
        out_ref[...] = (lax.dot_general(h, w2_ref[e], dn,
                                        preferred_element_type=jnp.float32)
                        + b2_ref[e]) * w16_ref[:, 0:1]


def _mlp(bmap, disp, W1, W3, W2, b1r, b3r, b2r, w16):
    grid_spec = pltpu.PrefetchScalarGridSpec(
        num_scalar_prefetch=1,
        grid=(NBLK,),
        in_specs=[
            pl.BlockSpec((BM, DIM), lambda b, m: (b, 0)),
            pl.BlockSpec((E, INTER, DIM), lambda b, m: (0, 0, 0)),
            pl.BlockSpec((E, INTER, DIM), lambda b, m: (0, 0, 0)),
            pl.BlockSpec((E, DIM, INTER), lambda b, m: (0, 0, 0)),
            pl.BlockSpec((E, 1, INTER), lambda b, m: (0, 0, 0)),
            pl.BlockSpec((E, 1, INTER), lambda b, m: (0, 0, 0)),
            pl.BlockSpec((E, 1, DIM), lambda b, m: (0, 0, 0)),
            pl.BlockSpec((BM, 128), lambda b, m: (b, 0)),
        ],
        out_specs=pl.BlockSpec(
            (BM, DIM), lambda b, m: (jnp.where(b < m[NBLK], b, NBLK), 0)),
    )
    return pl.pallas_call(
        _mlp_body,
        grid_spec=grid_spec,
        out_shape=jax.ShapeDtypeStruct((P + BM, DIM), jnp.float32),
    )(bmap, disp, W1, W3, W2, b1r, b3r, b2r, w16)


def _combine_body(out_hbm, pos_hbm, z_hbm, y_hbm,
                  idx_v, r0_v, r1_v, z_v, sem):
    wid = lax.axis_index("s") * 2 + lax.axis_index("c")
    base = wid * TPW
    pltpu.sync_copy(pos_hbm.at[wid], idx_v)

    for c in range(NCHK):
        g0 = pltpu.async_copy(out_hbm.at[idx_v.at[c]], r0_v, sem)
        g1 = pltpu.async_copy(out_hbm.at[idx_v.at[NCHK + c]], r1_v, sem)
        gz = pltpu.async_copy(z_hbm.at[pl.ds(base + c * CT, CT)], z_v, sem)
        g0.wait()
        g1.wait()
        gz.wait()

        def tok(i, _):
            def dchunk(jj, _):
                sl = pl.ds(jj * 16, 16)
                z_v[i, sl] = r0_v[i, sl] + r1_v[i, sl] + z_v[i, sl]
                return 0

            lax.fori_loop(0, DIM // 16, dchunk, 0, unroll=4)
            return 0

        lax.fori_loop(0, CT, tok, 0)
        pltpu.sync_copy(z_v, y_hbm.at[pl.ds(base + c * CT, CT)])


def _make_combine():
    mesh = plsc.VectorSubcoreMesh(core_axis_name="c", subcore_axis_name="s")
    return functools.partial(
        pl.kernel,
        out_type=jax.ShapeDtypeStruct((T, DIM), jnp.float32),
        mesh=mesh,
        scratch_types=[pltpu.VMEM((2 * NCHK, CT), jnp.int32),
                       pltpu.VMEM((CT, DIM), jnp.float32),
                       pltpu.VMEM((CT, DIM), jnp.float32),
                       pltpu.VMEM((CT, DIM), jnp.float32),
                       pltpu.SemaphoreType.DMA],
    )(_combine_body)


@jax.jit
def _moe(xf, Wg, bg, W1, b1r, W3, b3r, W2, b2r, Ws1, bs1, Ws3, bs3, Ws2, bs2):
    pos, wts, bmap = _gate(xf, Wg, bg)
    pos4 = pos.reshape(NW, TPW, 2).transpose(0, 2, 1) \
        .reshape(NW, 2, 2, 32).reshape(NW, 4, 32)
    wrep = jnp.broadcast_to(
        wts.reshape(NW, TPW, 2).transpose(0, 2, 1)
        .reshape(NW, 2, 2, 32).reshape(NW, 4, 32)[..., None],
        (NW, 4, 32, 128))
    disp, w16 = _make_dispatch()(xf, pos4, wrep)
    z = _shared(xf, Ws1, bs1, Ws3, bs3, Ws2, bs2)
    out = _mlp(bmap.reshape(NBLK + 1), disp, W1, W3, W2, b1r, b3r, b2r, w16)
    y = _make_combine()(out, pos4, z)
    return y


def kernel(x, Wg, bg, W1, b1, W3, b3, W2, b2, Ws1, bs1, Ws3, bs3, Ws2, bs2):
    shape = x.shape
    xf = x.reshape(-1, DIM)
    out = _moe(xf, Wg, bg.reshape(1, E), W1, b1.reshape(E, 1, INTER),
               W3, b3.reshape(E, 1, INTER), W2, b2.reshape(E, 1, DIM),
               Ws1, bs1.reshape(1, INTER), Ws3, bs3.reshape(1, INTER),
               Ws2, bs2.reshape(1, DIM))
    return out.reshape(shape)
